# Initial kernel scaffold; baseline (speedup 1.0000x reference)
import jax
import jax.numpy as jnp
from jax.experimental import pallas as pl


def kernel(x, edge_index, W1, b1, W2, b2):
    raise NotImplementedError("write your pallas kernel here")


# speedup vs baseline: 11.8369x; 11.8369x over previous
import jax
import jax.numpy as jnp
from jax import lax
from jax.experimental import pallas as pl
from jax.experimental.pallas import tpu as pltpu
from jax.experimental.pallas import tpu_sc as plsc

N = 10000
E = 320000
F = 128
C = 40
D2 = 128
NPAD = 10240
NC, NS, L = 2, 16, 16
NW = NC * NS
EPW = E // NW
K = 80
NCH = EPW // K
RPW = NPAD // NS
BR = 1024


def _sc_mesh():
    return plsc.VectorSubcoreMesh(core_axis_name="c", subcore_axis_name="s")


def _fill_rows(rows, d, val):
    v = jnp.full((L,), val, jnp.float32)

    def body(i, _):
        for j in range(d // L):
            rows[i, pl.ds(j * L, L)] = v
        return 0

    lax.fori_loop(0, K, body, 0)


def _zero_acc(rows, acc, d, s):
    _fill_rows(rows, d, 0.0)

    def body(i, _):
        pltpu.sync_copy(rows, acc.at[pl.ds(s * RPW + i * K, K)])
        return 0

    lax.fori_loop(0, RPW // K, body, 0)


def _drain_acc(acc, out_hbm, c, s):
    pltpu.sync_copy(acc.at[pl.ds(s * RPW, RPW)],
                    out_hbm.at[c, pl.ds(s * RPW, RPW)])


def _deg_body(dst_hbm, out_hbm, idx_d, rows, acc):
    c = lax.axis_index("c")
    s = lax.axis_index("s")
    w = s * NC + c
    _zero_acc(rows, acc, F, s)
    plsc.subcore_barrier()
    _fill_rows(rows, F, 1.0)

    def step(i, _):
        pltpu.sync_copy(dst_hbm.at[w * NCH + i], idx_d)
        pltpu.sync_copy(rows, acc.at[idx_d], add=True)
        return 0

    lax.fori_loop(0, NCH, step, 0)
    plsc.subcore_barrier()
    _drain_acc(acc, out_hbm, c, s)


def _deg_call(dst2):
    f = pl.kernel(
        _deg_body,
        out_type=jax.ShapeDtypeStruct((NC, NPAD, F), jnp.float32),
        mesh=_sc_mesh(),
        scratch_types=[
            pltpu.VMEM((K,), jnp.int32),
            pltpu.VMEM((K, F), jnp.float32),
            pltpu.VMEM_SHARED((NPAD, F), jnp.float32),
        ],
    )
    return f(dst2)


def _msg_body(xs_hbm, src_hbm, dst_hbm, out_hbm, idx_s, idx_d, rows, acc, sem):
    d = rows.shape[1]
    c = lax.axis_index("c")
    s = lax.axis_index("s")
    w = s * NC + c
    _zero_acc(rows, acc, d, s)
    plsc.subcore_barrier()

    def step(i, _):
        row = w * NCH + i
        pltpu.sync_copy(src_hbm.at[row], idx_s)
        pltpu.async_copy(xs_hbm.at[idx_s], rows, sem).wait()
        pltpu.sync_copy(dst_hbm.at[row], idx_d)
        pltpu.sync_copy(rows, acc.at[idx_d], add=True)
        return 0

    lax.fori_loop(0, NCH, step, 0)
    plsc.subcore_barrier()
    _drain_acc(acc, out_hbm, c, s)


def _msg_call(xs, src2, dst2, d):
    f = pl.kernel(
        _msg_body,
        out_type=jax.ShapeDtypeStruct((NC, NPAD, d), jnp.float32),
        mesh=_sc_mesh(),
        scratch_types=[
            pltpu.VMEM((K,), jnp.int32),
            pltpu.VMEM((K,), jnp.int32),
            pltpu.VMEM((K, d), jnp.float32),
            pltpu.VMEM_SHARED((NPAD, d), jnp.float32),
            pltpu.SemaphoreType.DMA,
        ],
    )
    return f(xs, src2, dst2)


def _dinv_of(dp):
    deg = 1.0 + dp[0, :, 0] + dp[1, :, 0]
    return lax.rsqrt(deg)


def _tc1_body(dp_ref, x_ref, w1_ref, o_ref):
    dinv = _dinv_of(dp_ref[...])
    h0 = jnp.dot(x_ref[...], w1_ref[...], preferred_element_type=jnp.float32)
    o_ref[...] = h0 * dinv[:, None]


def _tc1(degp, xpad, W1):
    return pl.pallas_call(
        _tc1_body,
        grid=(NPAD // BR,),
        in_specs=[
            pl.BlockSpec((NC, BR, F), lambda i: (0, i, 0)),
            pl.BlockSpec((BR, F), lambda i: (i, 0)),
            pl.BlockSpec((F, F), lambda i: (0, 0)),
        ],
        out_specs=pl.BlockSpec((BR, F), lambda i: (i, 0)),
        out_shape=jax.ShapeDtypeStruct((NPAD, F), jnp.float32),
    )(degp, xpad, W1)


def _tc2_body(dp_ref, a1_ref, xs_ref, w2_ref, b1_ref, o_ref):
    dinv = _dinv_of(dp_ref[...])
    a1 = a1_ref[...]
    sacc = a1[0] + a1[1] + xs_ref[...]
    h1 = jnp.maximum(dinv[:, None] * sacc + b1_ref[...], 0.0)
    z = jnp.dot(h1, w2_ref[...], preferred_element_type=jnp.float32)
    o_ref[...] = z * dinv[:, None]


def _tc2(degp, acc1, xs, w2p, b1r):
    return pl.pallas_call(
        _tc2_body,
        grid=(NPAD // BR,),
        in_specs=[
            pl.BlockSpec((NC, BR, F), lambda i: (0, i, 0)),
            pl.BlockSpec((NC, BR, F), lambda i: (0, i, 0)),
            pl.BlockSpec((BR, F), lambda i: (i, 0)),
            pl.BlockSpec((F, D2), lambda i: (0, 0)),
            pl.BlockSpec((1, F), lambda i: (0, 0)),
        ],
        out_specs=pl.BlockSpec((BR, D2), lambda i: (i, 0)),
        out_shape=jax.ShapeDtypeStruct((NPAD, D2), jnp.float32),
    )(degp, acc1, xs, w2p, b1r)


def _tc3_body(dp_ref, a2_ref, zs_ref, b2_ref, o_ref):
    dinv = _dinv_of(dp_ref[...])
    a2 = a2_ref[...]
    o = dinv[:, None] * (a2[0] + a2[1] + zs_ref[...]) + b2_ref[...]
    m = jnp.max(o, axis=1, keepdims=True)
    lo = o - m
    lse = jnp.log(jnp.sum(jnp.exp(lo), axis=1, keepdims=True))
    o_ref[...] = lo - lse


def _tc3(degp, acc2, zs, b2p):
    return pl.pallas_call(
        _tc3_body,
        grid=(NPAD // BR,),
        in_specs=[
            pl.BlockSpec((NC, BR, F), lambda i: (0, i, 0)),
            pl.BlockSpec((NC, BR, D2), lambda i: (0, i, 0)),
            pl.BlockSpec((BR, D2), lambda i: (i, 0)),
            pl.BlockSpec((1, D2), lambda i: (0, 0)),
        ],
        out_specs=pl.BlockSpec((BR, D2), lambda i: (i, 0)),
        out_shape=jax.ShapeDtypeStruct((NPAD, D2), jnp.float32),
    )(degp, acc2, zs, b2p)


def kernel(x, edge_index, W1, b1, W2, b2):
    src2 = edge_index[0].astype(jnp.int32).reshape(E // K, K)
    dst2 = edge_index[1].astype(jnp.int32).reshape(E // K, K)
    xpad = jnp.pad(x, ((0, NPAD - N), (0, 0)))
    w2p = jnp.pad(W2, ((0, 0), (0, D2 - C)))
    b1r = b1.reshape(1, F)
    b2p = jnp.concatenate(
        [b2, jnp.full((D2 - C,), -1e30, jnp.float32)]).reshape(1, D2)

    degp = _deg_call(dst2)
    xs = _tc1(degp, xpad, W1)
    acc1 = _msg_call(xs, src2, dst2, F)
    zs = _tc2(degp, acc1, xs, w2p, b1r)
    acc2 = _msg_call(zs, src2, dst2, D2)
    o = _tc3(degp, acc2, zs, b2p)
    return o[:N, :C]
